# Initial kernel scaffold; baseline (speedup 1.0000x reference)
import jax
import jax.numpy as jnp
from jax.experimental import pallas as pl


def kernel(outputs, targets):
    raise NotImplementedError("write your pallas kernel here")


# speedup vs baseline: 2.1736x; 2.1736x over previous
import functools

import jax
import jax.numpy as jnp
from jax.experimental import pallas as pl
from jax.experimental.pallas import tpu as pltpu

_C = 1000
_BETA = 0.9999
_BATCH = 16384
_R = 1024


def _wce_kernel(x_ref, t_ref, loss_ref, counts_acc, s_acc, *, n_steps):
    g = pl.program_id(0)

    @pl.when(g == 0)
    def _init():
        counts_acc[...] = jnp.zeros_like(counts_acc)
        s_acc[...] = jnp.zeros_like(s_acc)

    x = x_ref[...]
    t = t_ref[0, 0, :]

    m = jnp.max(x, axis=1, keepdims=True)
    s = jnp.sum(jnp.exp(x - m), axis=1, keepdims=True)
    lse = m[:, 0] + jnp.log(s[:, 0])

    lane = jax.lax.broadcasted_iota(jnp.int32, x.shape, 1)
    mask = (lane == t[:, None]).astype(jnp.float32)
    tgt = jnp.sum(x * mask, axis=1)
    nll = lse - tgt

    counts_acc[0, :] += jnp.sum(mask, axis=0)
    s_acc[0, :] += jnp.sum(mask * nll[:, None], axis=0)

    @pl.when(g == n_steps - 1)
    def _finish():
        counts = counts_acc[0, :]
        safe = jnp.maximum(counts, 1.0)
        w = (1.0 - _BETA) / (1.0 - jnp.exp(safe * jnp.log(_BETA)))
        num = jnp.sum(w * s_acc[0, :])
        den = jnp.sum(w * counts)
        loss_ref[...] = (num / den).reshape(1, 1)


def kernel(outputs, targets):
    n_steps = _BATCH // _R
    t3 = targets.reshape(n_steps, 1, _R)
    out = pl.pallas_call(
        functools.partial(_wce_kernel, n_steps=n_steps),
        grid=(n_steps,),
        in_specs=[
            pl.BlockSpec((_R, _C), lambda g: (g, 0)),
            pl.BlockSpec((1, 1, _R), lambda g: (g, 0, 0)),
        ],
        out_specs=pl.BlockSpec((1, 1), lambda g: (0, 0)),
        out_shape=jax.ShapeDtypeStruct((1, 1), jnp.float32),
        scratch_shapes=[
            pltpu.VMEM((1, _C), jnp.float32),
            pltpu.VMEM((1, _C), jnp.float32),
        ],
    )(outputs, t3)
    return out[0, 0]
